# baseline (speedup 1.0000x reference)
import jax
import jax.numpy as jnp
from jax.experimental import pallas as pl
from jax.experimental.pallas import tpu as pltpu
from jax.experimental.pallas import tpu_sc as plsc

BATCH = 16384
D = 64
GATHER_WINDOW = 128


def _sc_gather(user_table2, item_table2, uidx, iidx):
    mesh = plsc.VectorSubcoreMesh(core_axis_name="core", subcore_axis_name="subcore")
    out_t = jax.ShapeDtypeStruct((BATCH, 2 * D), user_table2.dtype)

    @pl.kernel(out_type=(out_t, out_t), mesh=mesh)
    def gather_kernel(ut_hbm, it_hbm, ui_hbm, ii_hbm, uo_hbm, io_hbm):
        def body(ui_vmem, ii_vmem, uo_vmem, io_vmem):
            pltpu.sync_copy(ut_hbm.at[ui_vmem.at[0]], uo_vmem)
            pltpu.sync_copy(it_hbm.at[ii_vmem.at[0]], io_vmem)

        pltpu.emit_pipeline(
            body,
            grid=(BATCH // GATHER_WINDOW,),
            in_specs=[
                pl.BlockSpec((1, GATHER_WINDOW), index_map=lambda i: (0, i)),
                pl.BlockSpec((1, GATHER_WINDOW), index_map=lambda i: (0, i)),
            ],
            out_specs=[
                pl.BlockSpec((GATHER_WINDOW, 2 * D), index_map=lambda i: (i, 0)),
                pl.BlockSpec((GATHER_WINDOW, 2 * D), index_map=lambda i: (i, 0)),
            ],
            core_axis_name=("core", "subcore"),
            dimension_semantics=(pltpu.PARALLEL,),
        )(ui_hbm, ii_hbm, uo_hbm, io_hbm)

    return gather_kernel(
        user_table2, item_table2, uidx.reshape(1, BATCH), iidx.reshape(1, BATCH)
    )


def _towers_kernel(ue2_ref, ie2_ref, upar_ref, ipar_ref,
                   uW1_ref, ub1_ref, uW2_ref, ub2_ref,
                   iW1_ref, ib1_ref, iW2_ref, ib2_ref, out_ref):
    f32 = jnp.float32

    def tower(e2, par, W1, b1, W2, b2):
        lo = e2[:, :D]
        e = lo + par * (e2[:, D:] - lo)
        h = jnp.maximum(jnp.dot(e, W1, preferred_element_type=f32) + b1, 0.0)
        o = jnp.dot(h, W2, preferred_element_type=f32) + b2
        norm = jnp.sqrt(jnp.sum(o * o, axis=1, keepdims=True))
        return o / jnp.maximum(norm, 1e-12)

    u = tower(ue2_ref[...], upar_ref[...], uW1_ref[...], ub1_ref[...],
              uW2_ref[...], ub2_ref[...])
    v = tower(ie2_ref[...], ipar_ref[...], iW1_ref[...], ib1_ref[...],
              iW2_ref[...], ib2_ref[...])
    out_ref[...] = jnp.sum(u * v, axis=1)


def _tc_towers(u_emb2, i_emb2, upar, ipar, uW1, ub1, uW2, ub2, iW1, ib1, iW2, ib2):
    B_BLK = 2048
    grid = (BATCH // B_BLK,)
    full = lambda shape: pl.BlockSpec(shape, lambda i: (0,) * len(shape))
    return pl.pallas_call(
        _towers_kernel,
        grid=grid,
        in_specs=[
            pl.BlockSpec((B_BLK, 2 * D), lambda i: (i, 0)),
            pl.BlockSpec((B_BLK, 2 * D), lambda i: (i, 0)),
            pl.BlockSpec((B_BLK, 1), lambda i: (i, 0)),
            pl.BlockSpec((B_BLK, 1), lambda i: (i, 0)),
            full((D, 2 * D)), full((1, 2 * D)), full((2 * D, D)), full((1, D)),
            full((D, 2 * D)), full((1, 2 * D)), full((2 * D, D)), full((1, D)),
        ],
        out_specs=pl.BlockSpec((B_BLK,), lambda i: (i,)),
        out_shape=jax.ShapeDtypeStruct((BATCH,), jnp.float32),
    )(u_emb2, i_emb2, upar, ipar,
      uW1, ub1.reshape(1, -1), uW2, ub2.reshape(1, -1),
      iW1, ib1.reshape(1, -1), iW2, ib2.reshape(1, -1))


def kernel(user_ids, item_ids, user_table, item_table,
           uW1, ub1, uW2, ub2, iW1, ib1, iW2, ib2):
    ut2 = user_table.reshape(-1, 2 * D)
    it2 = item_table.reshape(-1, 2 * D)
    uidx = jax.lax.shift_right_logical(user_ids, 1)
    iidx = jax.lax.shift_right_logical(item_ids, 1)
    upar = jax.lax.bitwise_and(user_ids, 1).astype(jnp.float32).reshape(BATCH, 1)
    ipar = jax.lax.bitwise_and(item_ids, 1).astype(jnp.float32).reshape(BATCH, 1)
    u_emb2, i_emb2 = _sc_gather(ut2, it2, uidx, iidx)
    return _tc_towers(u_emb2, i_emb2, upar, ipar,
                      uW1, ub1, uW2, ub2, iW1, ib1, iW2, ib2)


# speedup vs baseline: 1.0044x; 1.0044x over previous
import jax
import jax.numpy as jnp
from jax import lax
from jax.experimental import pallas as pl
from jax.experimental.pallas import tpu as pltpu
from jax.experimental.pallas import tpu_sc as plsc

BATCH = 16384
D = 64
NC, NS = 2, 16
NW = NC * NS
B_PER_W = BATCH // NW
CHUNK = 128
N_CHUNKS = B_PER_W // CHUNK


def _sc_gather(user_table, item_table, uids, iids):
    mesh = plsc.VectorSubcoreMesh(core_axis_name="c", subcore_axis_name="s")
    out_t = jax.ShapeDtypeStruct((BATCH, D), user_table.dtype)

    @pl.kernel(
        out_type=(out_t, out_t),
        mesh=mesh,
        scratch_types=[
            pltpu.VMEM((B_PER_W,), jnp.int32),
            pltpu.VMEM((B_PER_W,), jnp.int32),
            pltpu.VMEM((B_PER_W, D), jnp.float32),
            pltpu.VMEM((B_PER_W, D), jnp.float32),
            pltpu.SemaphoreType.DMA,
        ],
        compiler_params=pltpu.CompilerParams(use_tc_tiling_on_sc=False),
    )
    def gather_kernel(ut_hbm, it_hbm, ui_hbm, ii_hbm, uo_hbm, io_hbm,
                      idx_u, idx_i, rows_u, rows_i, sem):
        wid = lax.axis_index("s") * NC + lax.axis_index("c")
        base = wid * B_PER_W
        pltpu.sync_copy(ui_hbm.at[pl.ds(base, B_PER_W)], idx_u)
        pltpu.sync_copy(ii_hbm.at[pl.ds(base, B_PER_W)], idx_i)
        copies = []
        for c in range(N_CHUNKS):
            sl = pl.ds(c * CHUNK, CHUNK)
            copies.append(pltpu.async_copy(
                ut_hbm.at[idx_u.at[sl]], rows_u.at[sl], sem))
            copies.append(pltpu.async_copy(
                it_hbm.at[idx_i.at[sl]], rows_i.at[sl], sem))
        for cp in copies:
            cp.wait()
        pltpu.sync_copy(rows_u, uo_hbm.at[pl.ds(base, B_PER_W)])
        pltpu.sync_copy(rows_i, io_hbm.at[pl.ds(base, B_PER_W)])

    return gather_kernel(user_table, item_table, uids, iids)


def _towers_kernel(ue_ref, ie_ref, uW1_ref, ub1_ref, uW2_ref, ub2_ref,
                   iW1_ref, ib1_ref, iW2_ref, ib2_ref, out_ref):
    f32 = jnp.float32

    def tower(e, W1, b1, W2, b2):
        h = jnp.maximum(jnp.dot(e, W1, preferred_element_type=f32) + b1, 0.0)
        o = jnp.dot(h, W2, preferred_element_type=f32) + b2
        norm = jnp.sqrt(jnp.sum(o * o, axis=1, keepdims=True))
        return o / jnp.maximum(norm, 1e-12)

    u = tower(ue_ref[...], uW1_ref[...], ub1_ref[...], uW2_ref[...], ub2_ref[...])
    v = tower(ie_ref[...], iW1_ref[...], ib1_ref[...], iW2_ref[...], ib2_ref[...])
    out_ref[...] = jnp.sum(u * v, axis=1)


def _tc_towers(u_emb, i_emb, uW1, ub1, uW2, ub2, iW1, ib1, iW2, ib2):
    B_BLK = 2048
    grid = (BATCH // B_BLK,)
    full = lambda shape: pl.BlockSpec(shape, lambda i: (0,) * len(shape))
    return pl.pallas_call(
        _towers_kernel,
        grid=grid,
        in_specs=[
            pl.BlockSpec((B_BLK, D), lambda i: (i, 0)),
            pl.BlockSpec((B_BLK, D), lambda i: (i, 0)),
            full((D, 2 * D)), full((1, 2 * D)), full((2 * D, D)), full((1, D)),
            full((D, 2 * D)), full((1, 2 * D)), full((2 * D, D)), full((1, D)),
        ],
        out_specs=pl.BlockSpec((B_BLK,), lambda i: (i,)),
        out_shape=jax.ShapeDtypeStruct((BATCH,), jnp.float32),
    )(u_emb, i_emb,
      uW1, ub1.reshape(1, -1), uW2, ub2.reshape(1, -1),
      iW1, ib1.reshape(1, -1), iW2, ib2.reshape(1, -1))


def kernel(user_ids, item_ids, user_table, item_table,
           uW1, ub1, uW2, ub2, iW1, ib1, iW2, ib2):
    u_emb, i_emb = _sc_gather(user_table, item_table, user_ids, item_ids)
    return _tc_towers(u_emb, i_emb, uW1, ub1, uW2, ub2, iW1, ib1, iW2, ib2)


# speedup vs baseline: 1.5506x; 1.5437x over previous
import jax
import jax.numpy as jnp
from jax import lax
from jax.experimental import pallas as pl
from jax.experimental.pallas import tpu as pltpu
from jax.experimental.pallas import tpu_sc as plsc

BATCH = 16384
D = 64
NC, NS = 2, 16
NW = NC * NS
B_PER_W = BATCH // NW
CHUNK = 128
N_CHUNKS = B_PER_W // CHUNK
PHALF = 1 << 19
HBLK = 2048


def _retile_kernel(a_ref, b_ref, dst_ref):
    dst_ref[...] = jnp.concatenate([a_ref[...].T, b_ref[...].T], axis=1)


def _tc_retile(table_t):
    return pl.pallas_call(
        _retile_kernel,
        grid=(PHALF // HBLK,),
        in_specs=[
            pl.BlockSpec((D, HBLK), lambda j: (0, j)),
            pl.BlockSpec(
                (D, HBLK),
                lambda j: (0, jnp.minimum(j + PHALF // HBLK,
                                          (1000000 - 1) // HBLK))),
        ],
        out_specs=pl.BlockSpec((HBLK, 2 * D), lambda j: (j, 0)),
        out_shape=jax.ShapeDtypeStruct((PHALF, 2 * D), jnp.float32),
    )(table_t, table_t)


def _sc_gather(user_table2, item_table2, uidx, iidx):
    mesh = plsc.VectorSubcoreMesh(core_axis_name="c", subcore_axis_name="s")
    out_t = jax.ShapeDtypeStruct((BATCH, 2 * D), jnp.float32)

    @pl.kernel(
        out_type=(out_t, out_t),
        mesh=mesh,
        scratch_types=[
            pltpu.VMEM((B_PER_W,), jnp.int32),
            pltpu.VMEM((B_PER_W,), jnp.int32),
            pltpu.VMEM((B_PER_W, 2 * D), jnp.float32),
            pltpu.SemaphoreType.DMA,
        ],
        compiler_params=pltpu.CompilerParams(use_tc_tiling_on_sc=False),
    )
    def gather_kernel(ut_hbm, it_hbm, ui_hbm, ii_hbm, uo_hbm, io_hbm,
                      idx_u, idx_i, rows, sem):
        wid = lax.axis_index("s") * NC + lax.axis_index("c")
        base = wid * B_PER_W
        pltpu.sync_copy(ui_hbm.at[pl.ds(base, B_PER_W)], idx_u)
        pltpu.sync_copy(ii_hbm.at[pl.ds(base, B_PER_W)], idx_i)
        for tbl, idx, out in ((ut_hbm, idx_u, uo_hbm), (it_hbm, idx_i, io_hbm)):
            copies = []
            for c in range(N_CHUNKS):
                sl = pl.ds(c * CHUNK, CHUNK)
                copies.append(pltpu.async_copy(
                    tbl.at[idx.at[sl]], rows.at[sl], sem))
            for cp in copies:
                cp.wait()
            pltpu.sync_copy(rows, out.at[pl.ds(base, B_PER_W)])

    return gather_kernel(user_table2, item_table2, uidx, iidx)


def _towers_kernel(ue2_ref, ie2_ref, upar_ref, ipar_ref,
                   uW1_ref, ub1_ref, uW2_ref, ub2_ref,
                   iW1_ref, ib1_ref, iW2_ref, ib2_ref, out_ref):
    f32 = jnp.float32

    def tower(e2, par, W1, b1, W2, b2):
        lo = e2[:, :D]
        e = lo + par * (e2[:, D:] - lo)
        h = jnp.maximum(jnp.dot(e, W1, preferred_element_type=f32) + b1, 0.0)
        o = jnp.dot(h, W2, preferred_element_type=f32) + b2
        norm = jnp.sqrt(jnp.sum(o * o, axis=1, keepdims=True))
        return o / jnp.maximum(norm, 1e-12)

    u = tower(ue2_ref[...], upar_ref[...], uW1_ref[...], ub1_ref[...],
              uW2_ref[...], ub2_ref[...])
    v = tower(ie2_ref[...], ipar_ref[...], iW1_ref[...], ib1_ref[...],
              iW2_ref[...], ib2_ref[...])
    out_ref[...] = jnp.sum(u * v, axis=1)


def _tc_towers(u_emb2, i_emb2, upar, ipar, uW1, ub1, uW2, ub2, iW1, ib1, iW2, ib2):
    B_BLK = 2048
    grid = (BATCH // B_BLK,)
    full = lambda shape: pl.BlockSpec(shape, lambda i: (0,) * len(shape))
    return pl.pallas_call(
        _towers_kernel,
        grid=grid,
        in_specs=[
            pl.BlockSpec((B_BLK, 2 * D), lambda i: (i, 0)),
            pl.BlockSpec((B_BLK, 2 * D), lambda i: (i, 0)),
            pl.BlockSpec((B_BLK, 1), lambda i: (i, 0)),
            pl.BlockSpec((B_BLK, 1), lambda i: (i, 0)),
            full((D, 2 * D)), full((1, 2 * D)), full((2 * D, D)), full((1, D)),
            full((D, 2 * D)), full((1, 2 * D)), full((2 * D, D)), full((1, D)),
        ],
        out_specs=pl.BlockSpec((B_BLK,), lambda i: (i,)),
        out_shape=jax.ShapeDtypeStruct((BATCH,), jnp.float32),
    )(u_emb2, i_emb2, upar, ipar,
      uW1, ub1.reshape(1, -1), uW2, ub2.reshape(1, -1),
      iW1, ib1.reshape(1, -1), iW2, ib2.reshape(1, -1))


def kernel(user_ids, item_ids, user_table, item_table,
           uW1, ub1, uW2, ub2, iW1, ib1, iW2, ib2):
    ut2 = _tc_retile(user_table.T)
    it2 = _tc_retile(item_table.T)
    uidx = jax.lax.bitwise_and(user_ids, PHALF - 1)
    iidx = jax.lax.bitwise_and(item_ids, PHALF - 1)
    upar = jax.lax.shift_right_logical(user_ids, 19).astype(jnp.float32).reshape(BATCH, 1)
    ipar = jax.lax.shift_right_logical(item_ids, 19).astype(jnp.float32).reshape(BATCH, 1)
    u_emb2, i_emb2 = _sc_gather(ut2, it2, uidx, iidx)
    return _tc_towers(u_emb2, i_emb2, upar, ipar,
                      uW1, ub1, uW2, ub2, iW1, ib1, iW2, ib2)


# speedup vs baseline: 2.5683x; 1.6563x over previous
import jax
import jax.numpy as jnp
from jax import lax
from jax.experimental import pallas as pl
from jax.experimental.pallas import tpu as pltpu
from jax.experimental.pallas import tpu_sc as plsc

BATCH = 16384
D = 64
NC, NS = 2, 16
NW = NC * NS
B_PER_W = BATCH // NW
CHUNK = 128
N_CHUNKS = B_PER_W // CHUNK
QSPLIT = 1 << 18
HBLK = 8192
LASTBLK = (1000000 - 1) // HBLK


def _retile_kernel(a_ref, b_ref, c_ref, d_ref, dst_ref):
    eye = jnp.eye(D, dtype=jnp.float32)
    dims = (((0,), (0,)), ((), ()))
    u32 = jnp.uint32

    def t(x):
        return jax.lax.dot_general(x, eye, dims,
                                   preferred_element_type=jnp.float32)

    def pack(hi_f, lo_f):
        hi = jax.lax.bitcast_convert_type(hi_f, u32) + u32(0x8000)
        lo = jax.lax.bitcast_convert_type(lo_f, u32) + u32(0x8000)
        return jax.lax.bitcast_convert_type(
            (hi & u32(0xFFFF0000)) | (lo >> 16), jnp.float32)

    dst_ref[...] = jnp.concatenate(
        [pack(t(a_ref[...]), t(b_ref[...])),
         pack(t(c_ref[...]), t(d_ref[...]))], axis=1)


def _tc_retile(table_t):
    step = QSPLIT // HBLK

    def spec(k):
        return pl.BlockSpec(
            (D, HBLK), lambda j, k=k: (0, jnp.minimum(j + k * step, LASTBLK)))

    return pl.pallas_call(
        _retile_kernel,
        grid=(step,),
        in_specs=[spec(0), spec(1), spec(2), spec(3)],
        out_specs=pl.BlockSpec((HBLK, 2 * D), lambda j: (j, 0)),
        out_shape=jax.ShapeDtypeStruct((QSPLIT, 2 * D), jnp.float32),
        compiler_params=pltpu.CompilerParams(
            dimension_semantics=("parallel",)),
    )(table_t, table_t, table_t, table_t)


def _sc_gather(user_table2, item_table2, uidx, iidx):
    mesh = plsc.VectorSubcoreMesh(core_axis_name="c", subcore_axis_name="s")
    out_t = jax.ShapeDtypeStruct((BATCH, 2 * D), jnp.float32)

    @pl.kernel(
        out_type=(out_t, out_t),
        mesh=mesh,
        scratch_types=[
            pltpu.VMEM((B_PER_W,), jnp.int32),
            pltpu.VMEM((B_PER_W,), jnp.int32),
            pltpu.VMEM((B_PER_W, 2 * D), jnp.float32),
            pltpu.SemaphoreType.DMA,
        ],
        compiler_params=pltpu.CompilerParams(use_tc_tiling_on_sc=False),
    )
    def gather_kernel(ut_hbm, it_hbm, ui_hbm, ii_hbm, uo_hbm, io_hbm,
                      idx_u, idx_i, rows, sem):
        wid = lax.axis_index("s") * NC + lax.axis_index("c")
        base = wid * B_PER_W
        pltpu.sync_copy(ui_hbm.at[pl.ds(base, B_PER_W)], idx_u)
        pltpu.sync_copy(ii_hbm.at[pl.ds(base, B_PER_W)], idx_i)
        for tbl, idx, out in ((ut_hbm, idx_u, uo_hbm), (it_hbm, idx_i, io_hbm)):
            copies = []
            for c in range(N_CHUNKS):
                sl = pl.ds(c * CHUNK, CHUNK)
                copies.append(pltpu.async_copy(
                    tbl.at[idx.at[sl]], rows.at[sl], sem))
            for cp in copies:
                cp.wait()
            pltpu.sync_copy(rows, out.at[pl.ds(base, B_PER_W)])

    return gather_kernel(user_table2, item_table2, uidx, iidx)


def _towers_kernel(ue2_ref, ie2_ref, uw_ref, ul_ref, iw_ref, il_ref,
                   uW1_ref, ub1_ref, uW2_ref, ub2_ref,
                   iW1_ref, ib1_ref, iW2_ref, ib2_ref, out_ref):
    f32 = jnp.float32

    def tower(e2, p_word, p_lane, W1, b1, W2, b2):
        u32 = jnp.uint32
        bits = jax.lax.bitcast_convert_type(e2, u32)
        v_hi = jax.lax.bitcast_convert_type(bits & u32(0xFFFF0000), f32)
        v_lo = jax.lax.bitcast_convert_type(bits << 16, f32)
        v = v_hi + p_word * (v_lo - v_hi)
        e = v[:, :D] + p_lane * (v[:, D:] - v[:, :D])
        h = jnp.maximum(jnp.dot(e, W1, preferred_element_type=f32) + b1, 0.0)
        o = jnp.dot(h, W2, preferred_element_type=f32) + b2
        norm = jnp.sqrt(jnp.sum(o * o, axis=1, keepdims=True))
        return o / jnp.maximum(norm, 1e-12)

    u = tower(ue2_ref[...], uw_ref[...], ul_ref[...], uW1_ref[...],
              ub1_ref[...], uW2_ref[...], ub2_ref[...])
    v = tower(ie2_ref[...], iw_ref[...], il_ref[...], iW1_ref[...],
              ib1_ref[...], iW2_ref[...], ib2_ref[...])
    out_ref[...] = jnp.sum(u * v, axis=1)


def _tc_towers(u_emb2, i_emb2, uw, ul, iw, il,
               uW1, ub1, uW2, ub2, iW1, ib1, iW2, ib2):
    B_BLK = 2048
    grid = (BATCH // B_BLK,)
    full = lambda shape: pl.BlockSpec(shape, lambda i: (0,) * len(shape))
    par_spec = pl.BlockSpec((B_BLK, 1), lambda i: (i, 0))
    return pl.pallas_call(
        _towers_kernel,
        grid=grid,
        in_specs=[
            pl.BlockSpec((B_BLK, 2 * D), lambda i: (i, 0)),
            pl.BlockSpec((B_BLK, 2 * D), lambda i: (i, 0)),
            par_spec, par_spec, par_spec, par_spec,
            full((D, 2 * D)), full((1, 2 * D)), full((2 * D, D)), full((1, D)),
            full((D, 2 * D)), full((1, 2 * D)), full((2 * D, D)), full((1, D)),
        ],
        out_specs=pl.BlockSpec((B_BLK,), lambda i: (i,)),
        out_shape=jax.ShapeDtypeStruct((BATCH,), jnp.float32),
        compiler_params=pltpu.CompilerParams(
            dimension_semantics=("parallel",)),
    )(u_emb2, i_emb2, uw, ul, iw, il,
      uW1, ub1.reshape(1, -1), uW2, ub2.reshape(1, -1),
      iW1, ib1.reshape(1, -1), iW2, ib2.reshape(1, -1))


def kernel(user_ids, item_ids, user_table, item_table,
           uW1, ub1, uW2, ub2, iW1, ib1, iW2, ib2):
    ut2 = _tc_retile(user_table.T)
    it2 = _tc_retile(item_table.T)
    uidx = jax.lax.bitwise_and(user_ids, QSPLIT - 1)
    iidx = jax.lax.bitwise_and(item_ids, QSPLIT - 1)

    def par_bits(ids):
        sel = jax.lax.shift_right_logical(ids, 18)
        word = jax.lax.bitwise_and(sel, 1).astype(jnp.float32).reshape(BATCH, 1)
        lane = jax.lax.shift_right_logical(sel, 1).astype(jnp.float32).reshape(BATCH, 1)
        return word, lane

    uw, ul = par_bits(user_ids)
    iw, il = par_bits(item_ids)
    u_emb2, i_emb2 = _sc_gather(ut2, it2, uidx, iidx)
    return _tc_towers(u_emb2, i_emb2, uw, ul, iw, il,
                      uW1, ub1, uW2, ub2, iW1, ib1, iW2, ib2)


# speedup vs baseline: 2.5821x; 1.0054x over previous
import jax
import jax.numpy as jnp
from jax import lax
from jax.experimental import pallas as pl
from jax.experimental.pallas import tpu as pltpu
from jax.experimental.pallas import tpu_sc as plsc

BATCH = 16384
D = 64
NC, NS = 2, 16
NW = NC * NS
B_PER_W = BATCH // NW
CHUNK = 128
N_CHUNKS = B_PER_W // CHUNK
QSPLIT = 1 << 18
HBLK = 8192
LASTBLK = (1000000 - 1) // HBLK


def _retile_kernel(a_ref, b_ref, c_ref, d_ref, dst_ref):
    eye = jnp.eye(D, dtype=jnp.float32)
    dims = (((0,), (0,)), ((), ()))
    u32 = jnp.uint32

    def t(x):
        return jax.lax.dot_general(x, eye, dims,
                                   preferred_element_type=jnp.float32)

    def pack(hi_f, lo_f):
        hi = jax.lax.bitcast_convert_type(hi_f, u32) + u32(0x8000)
        lo = jax.lax.bitcast_convert_type(lo_f, u32) + u32(0x8000)
        return jax.lax.bitcast_convert_type(
            (hi & u32(0xFFFF0000)) | (lo >> 16), jnp.float32)

    dst_ref[...] = jnp.concatenate(
        [pack(t(a_ref[...]), t(b_ref[...])),
         pack(t(c_ref[...]), t(d_ref[...]))], axis=1)


def _tc_retile(table_t):
    step = QSPLIT // HBLK

    def spec(k):
        return pl.BlockSpec(
            (D, HBLK), lambda j, k=k: (0, jnp.minimum(j + k * step, LASTBLK)))

    return pl.pallas_call(
        _retile_kernel,
        grid=(step,),
        in_specs=[spec(0), spec(1), spec(2), spec(3)],
        out_specs=pl.BlockSpec((HBLK, 2 * D), lambda j: (j, 0)),
        out_shape=jax.ShapeDtypeStruct((QSPLIT, 2 * D), jnp.float32),
        compiler_params=pltpu.CompilerParams(
            dimension_semantics=("parallel",)),
    )(table_t, table_t, table_t, table_t)


def _sc_gather_one(table2, idx_arr):
    mesh = plsc.VectorSubcoreMesh(core_axis_name="c", subcore_axis_name="s")
    out_t = jax.ShapeDtypeStruct((BATCH, 2 * D), jnp.float32)

    @pl.kernel(
        out_type=out_t,
        mesh=mesh,
        scratch_types=[
            pltpu.VMEM((B_PER_W,), jnp.int32),
            pltpu.VMEM((B_PER_W, 2 * D), jnp.float32),
            pltpu.SemaphoreType.DMA,
        ],
        compiler_params=pltpu.CompilerParams(use_tc_tiling_on_sc=False),
    )
    def gather_kernel(t_hbm, i_hbm, o_hbm, idx, rows, sem):
        wid = lax.axis_index("s") * NC + lax.axis_index("c")
        base = wid * B_PER_W
        pltpu.sync_copy(i_hbm.at[pl.ds(base, B_PER_W)], idx)
        copies = []
        for c in range(N_CHUNKS):
            sl = pl.ds(c * CHUNK, CHUNK)
            copies.append(pltpu.async_copy(
                t_hbm.at[idx.at[sl]], rows.at[sl], sem))
        for cp in copies:
            cp.wait()
        pltpu.sync_copy(rows, o_hbm.at[pl.ds(base, B_PER_W)])

    return gather_kernel(table2, idx_arr)


def _towers_kernel(ue2_ref, ie2_ref, uw_ref, ul_ref, iw_ref, il_ref,
                   uW1_ref, ub1_ref, uW2_ref, ub2_ref,
                   iW1_ref, ib1_ref, iW2_ref, ib2_ref, out_ref):
    f32 = jnp.float32

    def tower(e2, p_word, p_lane, W1, b1, W2, b2):
        u32 = jnp.uint32
        bits = jax.lax.bitcast_convert_type(e2, u32)
        v_hi = jax.lax.bitcast_convert_type(bits & u32(0xFFFF0000), f32)
        v_lo = jax.lax.bitcast_convert_type(bits << 16, f32)
        v = v_hi + p_word * (v_lo - v_hi)
        e = v[:, :D] + p_lane * (v[:, D:] - v[:, :D])
        h = jnp.maximum(jnp.dot(e, W1, preferred_element_type=f32) + b1, 0.0)
        o = jnp.dot(h, W2, preferred_element_type=f32) + b2
        norm = jnp.sqrt(jnp.sum(o * o, axis=1, keepdims=True))
        return o / jnp.maximum(norm, 1e-12)

    u = tower(ue2_ref[...], uw_ref[...], ul_ref[...], uW1_ref[...],
              ub1_ref[...], uW2_ref[...], ub2_ref[...])
    v = tower(ie2_ref[...], iw_ref[...], il_ref[...], iW1_ref[...],
              ib1_ref[...], iW2_ref[...], ib2_ref[...])
    out_ref[...] = jnp.sum(u * v, axis=1)


def _tc_towers(u_emb2, i_emb2, uw, ul, iw, il,
               uW1, ub1, uW2, ub2, iW1, ib1, iW2, ib2):
    B_BLK = 2048
    grid = (BATCH // B_BLK,)
    full = lambda shape: pl.BlockSpec(shape, lambda i: (0,) * len(shape))
    par_spec = pl.BlockSpec((B_BLK, 1), lambda i: (i, 0))
    return pl.pallas_call(
        _towers_kernel,
        grid=grid,
        in_specs=[
            pl.BlockSpec((B_BLK, 2 * D), lambda i: (i, 0)),
            pl.BlockSpec((B_BLK, 2 * D), lambda i: (i, 0)),
            par_spec, par_spec, par_spec, par_spec,
            full((D, 2 * D)), full((1, 2 * D)), full((2 * D, D)), full((1, D)),
            full((D, 2 * D)), full((1, 2 * D)), full((2 * D, D)), full((1, D)),
        ],
        out_specs=pl.BlockSpec((B_BLK,), lambda i: (i,)),
        out_shape=jax.ShapeDtypeStruct((BATCH,), jnp.float32),
        compiler_params=pltpu.CompilerParams(
            dimension_semantics=("parallel",)),
    )(u_emb2, i_emb2, uw, ul, iw, il,
      uW1, ub1.reshape(1, -1), uW2, ub2.reshape(1, -1),
      iW1, ib1.reshape(1, -1), iW2, ib2.reshape(1, -1))


def kernel(user_ids, item_ids, user_table, item_table,
           uW1, ub1, uW2, ub2, iW1, ib1, iW2, ib2):
    ut2 = _tc_retile(user_table.T)
    it2 = _tc_retile(item_table.T)
    uidx = jax.lax.bitwise_and(user_ids, QSPLIT - 1)
    iidx = jax.lax.bitwise_and(item_ids, QSPLIT - 1)

    def par_bits(ids):
        sel = jax.lax.shift_right_logical(ids, 18)
        word = jax.lax.bitwise_and(sel, 1).astype(jnp.float32).reshape(BATCH, 1)
        lane = jax.lax.shift_right_logical(sel, 1).astype(jnp.float32).reshape(BATCH, 1)
        return word, lane

    uw, ul = par_bits(user_ids)
    iw, il = par_bits(item_ids)
    u_emb2 = _sc_gather_one(ut2, uidx)
    i_emb2 = _sc_gather_one(it2, iidx)
    return _tc_towers(u_emb2, i_emb2, uw, ul, iw, il,
                      uW1, ub1, uW2, ub2, iW1, ib1, iW2, ib2)
